# baseline (speedup 1.0000x reference)
import functools

import numpy as np
import jax
import jax.numpy as jnp
from jax import lax
from jax.experimental import pallas as pl
from jax.experimental.pallas import tpu as pltpu
from jax.experimental.pallas import tpu_sc as plsc

D_MODEL = 768
MAX_SEQ_LEN = 4096
BATCH = 4
SEQ_LEN = 4096
N_ROWS = BATCH * SEQ_LEN

NUM_CORES = 2
NUM_SUBCORES = 16
LANES = 16
NUM_WORKERS = NUM_CORES * NUM_SUBCORES
ROWS_PER_WORKER = N_ROWS // NUM_WORKERS
CHUNK = 32
NUM_CHUNKS = ROWS_PER_WORKER // CHUNK

SCALE = float(np.sqrt(np.float32(D_MODEL)))


def _sinusoidal_pe_np(max_seq_len, d_model):
    position = np.arange(0, max_seq_len, dtype=np.float32)[:, None]
    div_term = np.exp(
        np.arange(0, d_model, 2).astype(np.float32) * (-np.log(10000.0) / d_model)
    )
    pe = np.zeros((max_seq_len, d_model), dtype=np.float32)
    pe[:, 0::2] = np.sin(position * div_term)
    pe[:, 1::2] = np.cos(position * div_term)
    return pe


_PE = _sinusoidal_pe_np(MAX_SEQ_LEN, D_MODEL)


_MESH = plsc.VectorSubcoreMesh(core_axis_name="c", subcore_axis_name="s")


@functools.partial(
    pl.kernel,
    mesh=_MESH,
    out_type=jax.ShapeDtypeStruct((N_ROWS, D_MODEL), jnp.float32),
    scratch_types=[
        pltpu.VMEM((ROWS_PER_WORKER,), jnp.int32),
        pltpu.VMEM((2, CHUNK, D_MODEL), jnp.float32),
        pltpu.VMEM((3, CHUNK, D_MODEL), jnp.float32),
        pltpu.SemaphoreType.DMA,
        pltpu.SemaphoreType.DMA,
        pltpu.SemaphoreType.DMA,
    ],
)
def _embed_sc(x_hbm, table_hbm, pe_hbm, out_hbm,
              idx_v, rows_v, acc_v, gsem, psem, osem):
    wid = lax.axis_index("s") * NUM_CORES + lax.axis_index("c")
    base = wid * ROWS_PER_WORKER
    pos_base = base % SEQ_LEN

    pltpu.sync_copy(x_hbm.at[pl.ds(base, ROWS_PER_WORKER)], idx_v)

    def gather_copy(g, buf):
        return pltpu.make_async_copy(
            table_hbm.at[idx_v.at[pl.ds(g * CHUNK, CHUNK)]],
            rows_v.at[buf], gsem)

    def pe_copy(g, buf):
        return pltpu.make_async_copy(
            pe_hbm.at[pl.ds(pos_base + g * CHUNK, CHUNK)],
            acc_v.at[buf], psem)

    def out_copy(g, buf):
        return pltpu.make_async_copy(
            acc_v.at[buf], out_hbm.at[pl.ds(base + g * CHUNK, CHUNK)], osem)

    gather_copy(0, 0).start()
    pe_copy(0, 0).start()

    def chunk_body(g, carry):
        b = g % 2
        a = g % 3
        gather_copy(g, b).wait()

        @pl.when(g < NUM_CHUNKS - 1)
        def _():
            gather_copy(g + 1, (g + 1) % 2).start()

        pe_copy(g, a).wait()

        @pl.when(g >= 2)
        def _():
            out_copy(0, 0).wait()

        @pl.when(g < NUM_CHUNKS - 1)
        def _():
            pe_copy(g + 1, (g + 1) % 3).start()

        def row_body(r, c):
            for j in range(D_MODEL // LANES):
                v = rows_v[b, r, pl.ds(j * LANES, LANES)]
                plsc.addupdate(acc_v.at[a, r, pl.ds(j * LANES, LANES)],
                               v * SCALE)
            return c

        lax.fori_loop(0, CHUNK, row_body, 0)

        out_copy(g, a).start()
        return carry

    lax.fori_loop(0, NUM_CHUNKS, chunk_body, 0)

    out_copy(0, 0).wait()
    out_copy(0, 0).wait()


def kernel(x, table):
    xf = x.reshape(N_ROWS).astype(jnp.int32)
    pe = jnp.asarray(_PE)
    out = _embed_sc(xf, table, pe)
    return out.reshape(BATCH, SEQ_LEN, D_MODEL)


# speedup vs baseline: 1.7153x; 1.7153x over previous
import functools

import numpy as np
import jax
import jax.numpy as jnp
from jax import lax
from jax.experimental import pallas as pl
from jax.experimental.pallas import tpu as pltpu
from jax.experimental.pallas import tpu_sc as plsc

D_MODEL = 768
MAX_SEQ_LEN = 4096
BATCH = 4
SEQ_LEN = 4096
N_ROWS = BATCH * SEQ_LEN

NUM_CORES = 2
NUM_SUBCORES = 16
LANES = 16
NUM_WORKERS = NUM_CORES * NUM_SUBCORES
ROWS_PER_WORKER = N_ROWS // NUM_WORKERS
CHUNK = 32
NUM_CHUNKS = ROWS_PER_WORKER // CHUNK

SCALE = float(np.sqrt(np.float32(D_MODEL)))


def _sinusoidal_pe_np(max_seq_len, d_model):
    position = np.arange(0, max_seq_len, dtype=np.float32)[:, None]
    div_term = np.exp(
        np.arange(0, d_model, 2).astype(np.float32) * (-np.log(10000.0) / d_model)
    )
    pe = np.zeros((max_seq_len, d_model), dtype=np.float32)
    pe[:, 0::2] = np.sin(position * div_term)
    pe[:, 1::2] = np.cos(position * div_term)
    return pe


_PE = _sinusoidal_pe_np(MAX_SEQ_LEN, D_MODEL)


_MESH = plsc.VectorSubcoreMesh(core_axis_name="c", subcore_axis_name="s")


@functools.partial(
    pl.kernel,
    mesh=_MESH,
    out_type=jax.ShapeDtypeStruct((N_ROWS, D_MODEL), jnp.float32),
    scratch_types=[
        pltpu.VMEM((ROWS_PER_WORKER,), jnp.int32),
        pltpu.VMEM((2, CHUNK, D_MODEL), jnp.float32),
        pltpu.VMEM((3, CHUNK, D_MODEL), jnp.float32),
        pltpu.SemaphoreType.DMA,
        pltpu.SemaphoreType.DMA,
        pltpu.SemaphoreType.DMA,
    ],
)
def _embed_sc(x_hbm, table_hbm, pe_hbm, out_hbm,
              idx_v, rows_v, acc_v, gsem, psem, osem):
    wid = lax.axis_index("s") * NUM_CORES + lax.axis_index("c")
    base = wid * ROWS_PER_WORKER
    pos_base = base % SEQ_LEN

    pltpu.sync_copy(x_hbm.at[pl.ds(base, ROWS_PER_WORKER)], idx_v)

    def gather_copy(g, buf):
        return pltpu.make_async_copy(
            table_hbm.at[idx_v.at[pl.ds(g * CHUNK, CHUNK)]],
            rows_v.at[buf], gsem)

    def pe_copy(g, buf):
        return pltpu.make_async_copy(
            pe_hbm.at[pl.ds(pos_base + g * CHUNK, CHUNK)],
            acc_v.at[buf], psem)

    def out_copy(g, buf):
        return pltpu.make_async_copy(
            acc_v.at[buf], out_hbm.at[pl.ds(base + g * CHUNK, CHUNK)], osem)

    gather_copy(0, 0).start()
    pe_copy(0, 0).start()

    def chunk_body(g, carry):
        b = g % 2
        a = g % 3
        gather_copy(g, b).wait()

        @pl.when(g < NUM_CHUNKS - 1)
        def _():
            gather_copy(g + 1, (g + 1) % 2).start()

        pe_copy(g, a).wait()

        @pl.when(g >= 2)
        def _():
            out_copy(0, 0).wait()

        @pl.when(g < NUM_CHUNKS - 1)
        def _():
            pe_copy(g + 1, (g + 1) % 3).start()

        @plsc.parallel_loop(0, CHUNK, 1, unroll=2)
        def _(r):
            for j in range(D_MODEL // LANES):
                v = rows_v[b, r, pl.ds(j * LANES, LANES)]
                plsc.addupdate(acc_v.at[a, r, pl.ds(j * LANES, LANES)],
                               v * SCALE)

        out_copy(g, a).start()
        return carry

    lax.fori_loop(0, NUM_CHUNKS, chunk_body, 0)

    out_copy(0, 0).wait()
    out_copy(0, 0).wait()


def kernel(x, table):
    xf = x.reshape(N_ROWS).astype(jnp.int32)
    pe = jnp.asarray(_PE)
    out = _embed_sc(xf, table, pe)
    return out.reshape(BATCH, SEQ_LEN, D_MODEL)


# speedup vs baseline: 1.7154x; 1.0000x over previous
import functools

import numpy as np
import jax
import jax.numpy as jnp
from jax import lax
from jax.experimental import pallas as pl
from jax.experimental.pallas import tpu as pltpu
from jax.experimental.pallas import tpu_sc as plsc

D_MODEL = 768
MAX_SEQ_LEN = 4096
BATCH = 4
SEQ_LEN = 4096
N_ROWS = BATCH * SEQ_LEN

NUM_CORES = 2
NUM_SUBCORES = 16
LANES = 16
NUM_WORKERS = NUM_CORES * NUM_SUBCORES
ROWS_PER_WORKER = N_ROWS // NUM_WORKERS
CHUNK = 32
NUM_CHUNKS = ROWS_PER_WORKER // CHUNK

SCALE = float(np.sqrt(np.float32(D_MODEL)))


def _sinusoidal_pe_np(max_seq_len, d_model):
    position = np.arange(0, max_seq_len, dtype=np.float32)[:, None]
    div_term = np.exp(
        np.arange(0, d_model, 2).astype(np.float32) * (-np.log(10000.0) / d_model)
    )
    pe = np.zeros((max_seq_len, d_model), dtype=np.float32)
    pe[:, 0::2] = np.sin(position * div_term)
    pe[:, 1::2] = np.cos(position * div_term)
    return pe


_PE = _sinusoidal_pe_np(MAX_SEQ_LEN, D_MODEL)


_MESH = plsc.VectorSubcoreMesh(core_axis_name="c", subcore_axis_name="s")


@functools.partial(
    pl.kernel,
    mesh=_MESH,
    out_type=jax.ShapeDtypeStruct((N_ROWS, D_MODEL), jnp.float32),
    scratch_types=[
        pltpu.VMEM((ROWS_PER_WORKER,), jnp.int32),
        pltpu.VMEM((2, CHUNK, D_MODEL), jnp.float32),
        pltpu.VMEM((3, CHUNK, D_MODEL), jnp.float32),
        pltpu.SemaphoreType.DMA,
        pltpu.SemaphoreType.DMA,
        pltpu.SemaphoreType.DMA,
    ],
)
def _embed_sc(x_hbm, table_hbm, pe_hbm, out_hbm,
              idx_v, rows_v, acc_v, gsem, psem, osem):
    wid = lax.axis_index("s") * NUM_CORES + lax.axis_index("c")
    base = wid * ROWS_PER_WORKER
    pos_base = base % SEQ_LEN

    pltpu.sync_copy(x_hbm.at[pl.ds(base, ROWS_PER_WORKER)], idx_v)

    def gather_copy(g, buf):
        return pltpu.make_async_copy(
            table_hbm.at[idx_v.at[pl.ds(g * CHUNK, CHUNK)]],
            rows_v.at[buf], gsem)

    def pe_copy(g, buf):
        return pltpu.make_async_copy(
            pe_hbm.at[pl.ds(pos_base + g * CHUNK, CHUNK)],
            acc_v.at[buf], psem)

    def out_copy(g, buf):
        return pltpu.make_async_copy(
            acc_v.at[buf], out_hbm.at[pl.ds(base + g * CHUNK, CHUNK)], osem)

    gather_copy(0, 0).start()
    pe_copy(0, 0).start()

    def chunk_body(g, carry):
        b = g % 2
        a = g % 3
        gather_copy(g, b).wait()

        @pl.when(g < NUM_CHUNKS - 1)
        def _():
            gather_copy(g + 1, (g + 1) % 2).start()

        pe_copy(g, a).wait()

        @pl.when(g >= 2)
        def _():
            out_copy(0, 0).wait()

        @pl.when(g < NUM_CHUNKS - 1)
        def _():
            pe_copy(g + 1, (g + 1) % 3).start()

        @plsc.parallel_loop(0, CHUNK, 1, unroll=4)
        def _(r):
            for j in range(D_MODEL // LANES):
                v = rows_v[b, r, pl.ds(j * LANES, LANES)]
                plsc.addupdate(acc_v.at[a, r, pl.ds(j * LANES, LANES)],
                               v * SCALE)

        out_copy(g, a).start()
        return carry

    lax.fori_loop(0, NUM_CHUNKS, chunk_body, 0)

    out_copy(0, 0).wait()
    out_copy(0, 0).wait()


def kernel(x, table):
    xf = x.reshape(N_ROWS).astype(jnp.int32)
    pe = jnp.asarray(_PE)
    out = _embed_sc(xf, table, pe)
    return out.reshape(BATCH, SEQ_LEN, D_MODEL)


# speedup vs baseline: 1.7374x; 1.0129x over previous
import functools

import numpy as np
import jax
import jax.numpy as jnp
from jax import lax
from jax.experimental import pallas as pl
from jax.experimental.pallas import tpu as pltpu
from jax.experimental.pallas import tpu_sc as plsc

D_MODEL = 768
MAX_SEQ_LEN = 4096
BATCH = 4
SEQ_LEN = 4096
N_ROWS = BATCH * SEQ_LEN

NUM_CORES = 2
NUM_SUBCORES = 16
LANES = 16
NUM_WORKERS = NUM_CORES * NUM_SUBCORES
POS_PER_WORKER = SEQ_LEN // NUM_WORKERS
CHUNK = 16
NUM_CHUNKS = POS_PER_WORKER // CHUNK

SCALE = float(np.sqrt(np.float32(D_MODEL)))


def _sinusoidal_pe_np(max_seq_len, d_model):
    position = np.arange(0, max_seq_len, dtype=np.float32)[:, None]
    div_term = np.exp(
        np.arange(0, d_model, 2).astype(np.float32) * (-np.log(10000.0) / d_model)
    )
    pe = np.zeros((max_seq_len, d_model), dtype=np.float32)
    pe[:, 0::2] = np.sin(position * div_term)
    pe[:, 1::2] = np.cos(position * div_term)
    return pe


_PE = _sinusoidal_pe_np(MAX_SEQ_LEN, D_MODEL)


_MESH = plsc.VectorSubcoreMesh(core_axis_name="c", subcore_axis_name="s")


@functools.partial(
    pl.kernel,
    mesh=_MESH,
    out_type=jax.ShapeDtypeStruct((N_ROWS, D_MODEL), jnp.float32),
    scratch_types=[
        pltpu.VMEM((BATCH, POS_PER_WORKER), jnp.int32),
        pltpu.VMEM((2, BATCH, CHUNK, D_MODEL), jnp.float32),
        pltpu.VMEM((2, CHUNK, D_MODEL), jnp.float32),
        pltpu.SemaphoreType.DMA,
        pltpu.SemaphoreType.DMA,
        pltpu.SemaphoreType.DMA,
    ],
)
def _embed_sc(x_hbm, table_hbm, pe_hbm, out_hbm,
              idx_v, rows_v, pe_v, gsem, psem, osem):
    wid = lax.axis_index("s") * NUM_CORES + lax.axis_index("c")
    pos0 = wid * POS_PER_WORKER

    for b in range(BATCH):
        pltpu.sync_copy(
            x_hbm.at[pl.ds(b * SEQ_LEN + pos0, POS_PER_WORKER)], idx_v.at[b])

    def gather_copy(g, buf, b):
        return pltpu.make_async_copy(
            table_hbm.at[idx_v.at[b, pl.ds(g * CHUNK, CHUNK)]],
            rows_v.at[buf, b], gsem)

    def pe_copy(g, buf):
        return pltpu.make_async_copy(
            pe_hbm.at[pl.ds(pos0 + g * CHUNK, CHUNK)], pe_v.at[buf], psem)

    def out_copy(g, buf, b):
        return pltpu.make_async_copy(
            rows_v.at[buf, b],
            out_hbm.at[pl.ds(b * SEQ_LEN + pos0 + g * CHUNK, CHUNK)], osem)

    for b in range(BATCH):
        gather_copy(0, 0, b).start()
    pe_copy(0, 0).start()

    def chunk_body(g, carry):
        buf = g % 2
        for b in range(BATCH):
            gather_copy(g, buf, b).wait()
        pe_copy(g, buf).wait()

        @pl.when(g < NUM_CHUNKS - 1)
        def _():
            @pl.when(g >= 1)
            def _():
                for b in range(BATCH):
                    out_copy(0, 0, 0).wait()
            for b in range(BATCH):
                gather_copy(g + 1, 1 - buf, b).start()
            pe_copy(g + 1, 1 - buf).start()

        @plsc.parallel_loop(0, CHUNK, 1, unroll=2)
        def _(r):
            for j in range(D_MODEL // LANES):
                sl = pl.ds(j * LANES, LANES)
                pv = pe_v[buf, r, sl]
                for b in range(BATCH):
                    rows_v[buf, b, r, sl] = pv + rows_v[buf, b, r, sl] * SCALE

        for b in range(BATCH):
            out_copy(g, buf, b).start()
        return carry

    lax.fori_loop(0, NUM_CHUNKS, chunk_body, 0)

    for _ in range(2 * BATCH):
        out_copy(0, 0, 0).wait()


def kernel(x, table):
    xf = x.reshape(N_ROWS).astype(jnp.int32)
    pe = jnp.asarray(_PE)
    out = _embed_sc(xf, table, pe)
    return out.reshape(BATCH, SEQ_LEN, D_MODEL)
